# Initial kernel scaffold; baseline (speedup 1.0000x reference)
import jax
import jax.numpy as jnp
from jax.experimental import pallas as pl


def kernel(boxes, scores):
    raise NotImplementedError("write your pallas kernel here")


# speedup vs baseline: 19.1585x; 19.1585x over previous
import jax
import jax.numpy as jnp
from jax.experimental import pallas as pl
from jax.experimental.pallas import tpu as pltpu

_SCORE_THRESH = 0.05
_NMS_THRESH = 0.5
_MAX_DET = 100
_N = 20000
_R = 160
_C = 128
_PAD = _R * _C


def _nms_kernel(x1_ref, y1_ref, x2_ref, y2_ref, s_ref, out_ref, work_ref, area_ref):
    x1 = x1_ref[...]
    y1 = y1_ref[...]
    x2 = x2_ref[...]
    y2 = y2_ref[...]
    s = s_ref[...]
    area_ref[...] = (x2 - x1) * (y2 - y1)
    work_ref[...] = jnp.where(s > _SCORE_THRESH, s, -jnp.inf)

    flat_iota = (
        jax.lax.broadcasted_iota(jnp.int32, (_R, _C), 0) * _C
        + jax.lax.broadcasted_iota(jnp.int32, (_R, _C), 1)
    )
    lane = jax.lax.broadcasted_iota(jnp.int32, (1, _C), 1)

    def step(i, carry):
        w = work_ref[...]
        m = jnp.max(w)
        valid = m != -jnp.inf
        idx = jnp.min(jnp.where(w == m, flat_iota, _PAD))
        sel = flat_iota == idx
        bx1 = jnp.max(jnp.where(sel, x1, -jnp.inf))
        by1 = jnp.max(jnp.where(sel, y1, -jnp.inf))
        bx2 = jnp.max(jnp.where(sel, x2, -jnp.inf))
        by2 = jnp.max(jnp.where(sel, y2, -jnp.inf))
        xx1 = jnp.maximum(x1, bx1)
        yy1 = jnp.maximum(y1, by1)
        xx2 = jnp.minimum(x2, bx2)
        yy2 = jnp.minimum(y2, by2)
        inter = jnp.maximum(xx2 - xx1, 0.0) * jnp.maximum(yy2 - yy1, 0.0)
        barea = (bx2 - bx1) * (by2 - by1)
        iou = inter / (area_ref[...] + barea - inter + 1e-9)
        suppress = (iou > _NMS_THRESH) & valid
        work_ref[...] = jnp.where(suppress, -jnp.inf, w)
        row = (
            jnp.where(lane == 0, bx1, 0.0)
            + jnp.where(lane == 1, by1, 0.0)
            + jnp.where(lane == 2, bx2, 0.0)
            + jnp.where(lane == 3, by2, 0.0)
            + jnp.where(lane == 4, m, 0.0)
        )
        out_ref[pl.ds(i, 1), :] = jnp.where(valid, row, 0.0)
        return carry

    jax.lax.fori_loop(0, _MAX_DET, step, 0)


def kernel(boxes, scores):
    pad_boxes = jnp.zeros((_PAD - _N, 4), dtype=boxes.dtype)
    b = jnp.concatenate([boxes, pad_boxes], axis=0)
    s = jnp.concatenate(
        [scores, jnp.full((_PAD - _N,), -1.0, dtype=scores.dtype)], axis=0
    ).reshape(_R, _C)
    x1 = b[:, 0].reshape(_R, _C)
    y1 = b[:, 1].reshape(_R, _C)
    x2 = b[:, 2].reshape(_R, _C)
    y2 = b[:, 3].reshape(_R, _C)
    out = pl.pallas_call(
        _nms_kernel,
        out_shape=jax.ShapeDtypeStruct((_MAX_DET, _C), jnp.float32),
        scratch_shapes=[
            pltpu.VMEM((_R, _C), jnp.float32),
            pltpu.VMEM((_R, _C), jnp.float32),
        ],
    )(x1, y1, x2, y2, s)
    return out[:, :5]


# speedup vs baseline: 20.0093x; 1.0444x over previous
import jax
import jax.numpy as jnp
from jax.experimental import pallas as pl
from jax.experimental.pallas import tpu as pltpu

_SCORE_THRESH = 0.05
_NMS_THRESH = 0.5
_MAX_DET = 100
_N = 20000
_R = 160
_C = 128
_PAD = _R * _C


def _nms_kernel(x1_ref, y1_ref, x2_ref, y2_ref, s_ref, out_ref, work_ref, area_ref):
    x1 = x1_ref[...]
    y1 = y1_ref[...]
    x2 = x2_ref[...]
    y2 = y2_ref[...]
    s = s_ref[...]
    area_ref[...] = (x2 - x1) * (y2 - y1)
    w0 = jnp.where(s > _SCORE_THRESH, s, -jnp.inf)
    work_ref[...] = w0

    flat_iota = (
        jax.lax.broadcasted_iota(jnp.int32, (_R, _C), 0) * _C
        + jax.lax.broadcasted_iota(jnp.int32, (_R, _C), 1)
    )
    lane = jax.lax.broadcasted_iota(jnp.int32, (1, _C), 1)

    def locate(w, m):
        idx = jnp.min(jnp.where(w == m, flat_iota, _PAD))
        r = idx // _C
        sel = lane == idx % _C

        def pick(ref):
            rowv = ref[pl.ds(r, 1), :]
            return jnp.max(jnp.where(sel, rowv, -jnp.inf))

        return pick(x1_ref), pick(y1_ref), pick(x2_ref), pick(y2_ref)

    m0 = jnp.max(w0)
    best0 = locate(w0, m0)

    def step(i, carry):
        m, bx1, by1, bx2, by2 = carry
        valid = m != -jnp.inf
        row = (
            jnp.where(lane == 0, bx1, 0.0)
            + jnp.where(lane == 1, by1, 0.0)
            + jnp.where(lane == 2, bx2, 0.0)
            + jnp.where(lane == 3, by2, 0.0)
            + jnp.where(lane == 4, m, 0.0)
        )
        out_ref[pl.ds(i, 1), :] = jnp.where(valid, row, 0.0)
        w = work_ref[...]
        xx1 = jnp.maximum(x1, bx1)
        yy1 = jnp.maximum(y1, by1)
        xx2 = jnp.minimum(x2, bx2)
        yy2 = jnp.minimum(y2, by2)
        inter = jnp.maximum(xx2 - xx1, 0.0) * jnp.maximum(yy2 - yy1, 0.0)
        barea = (bx2 - bx1) * (by2 - by1)
        iou = inter / (area_ref[...] + barea - inter + 1e-9)
        suppress = (iou > _NMS_THRESH) & valid
        neww = jnp.where(suppress, -jnp.inf, w)
        work_ref[...] = neww
        nm = jnp.max(neww)
        nbx1, nby1, nbx2, nby2 = locate(neww, nm)
        return (nm, nbx1, nby1, nbx2, nby2)

    jax.lax.fori_loop(0, _MAX_DET, step, (m0,) + best0)


def kernel(boxes, scores):
    pad_boxes = jnp.zeros((_PAD - _N, 4), dtype=boxes.dtype)
    b = jnp.concatenate([boxes, pad_boxes], axis=0)
    s = jnp.concatenate(
        [scores, jnp.full((_PAD - _N,), -1.0, dtype=scores.dtype)], axis=0
    ).reshape(_R, _C)
    x1 = b[:, 0].reshape(_R, _C)
    y1 = b[:, 1].reshape(_R, _C)
    x2 = b[:, 2].reshape(_R, _C)
    y2 = b[:, 3].reshape(_R, _C)
    out = pl.pallas_call(
        _nms_kernel,
        out_shape=jax.ShapeDtypeStruct((_MAX_DET, _C), jnp.float32),
        scratch_shapes=[
            pltpu.VMEM((_R, _C), jnp.float32),
            pltpu.VMEM((_R, _C), jnp.float32),
        ],
    )(x1, y1, x2, y2, s)
    return out[:, :5]
